# baseline (speedup 1.0000x reference)
import functools

import jax
import jax.numpy as jnp
from jax import lax
from jax.experimental import pallas as pl
from jax.experimental.pallas import tpu as pltpu
from jax.experimental.pallas import tpu_sc as plsc

_B, _Q, _NT, _C = 8, 256, 32, 32
_WCE, _WBB, _WGI = 1.0, 5.0, 2.0
_L = 16


def _splat_i(v):
    return jnp.full((_L,), v, jnp.int32)


def _sc_body(
    x_hbm, pbT_hbm, tb_hbm, lbl_hbm, pcT_hbm, tcT_hbm, out_hbm,
    x_v, pbT_v, tb_v, lbl_v, pcT_v, tcT_v, cost_v, y_v, cbb_v, used_v,
    src_v, ext_v, stage_v, sem,
):
    w = lax.axis_index("s") * 2 + lax.axis_index("c")

    @pl.when(w < _B)
    def _work():
        b = w
        pltpu.sync_copy(x_hbm.at[b], x_v)
        pltpu.sync_copy(pbT_hbm.at[b], pbT_v)
        pltpu.sync_copy(tb_hbm.at[b], tb_v)
        pltpu.sync_copy(lbl_hbm.at[b], lbl_v)
        pltpu.sync_copy(pcT_hbm.at[b], pcT_v)
        pltpu.sync_copy(tcT_hbm.at[b], tcT_v)

        lane = lax.iota(jnp.int32, _L)
        lane0 = lane == 0
        zeros = jnp.zeros((_L,), jnp.float32)

        def _init(t, _):
            used_v[pl.ds(t * _L, _L)] = zeros
            return 0

        lax.fori_loop(0, _Q // _L, _init, 0)

        def _build_j(j, _):
            lblv = plsc.load_gather(lbl_v, [_splat_i(j)])
            tbv = [plsc.load_gather(tb_v, [_splat_i(j), _splat_i(dd)]) for dd in range(6)]

            def _build_q(t, _):
                qv = lane + t * _L
                y = plsc.load_gather(x_v, [qv, lblv])
                prob = 1.0 / (1.0 + jnp.exp(-y))
                cb = jnp.zeros((_L,), jnp.float32)
                for dd in range(6):
                    cb = cb + jnp.abs(pbT_v[dd, pl.ds(t * _L, _L)] - tbv[dd])
                cost_v[j, pl.ds(t * _L, _L)] = cb - prob
                y_v[j, pl.ds(t * _L, _L)] = y
                cbb_v[j, pl.ds(t * _L, _L)] = cb
                return 0

            lax.fori_loop(0, _Q // _L, _build_q, 0)
            return 0

        lax.fori_loop(0, _NT, _build_j, 0)

        def _ext_t(t, _):
            sl = pl.ds(t * _L, _L)
            for dd in range(3):
                lo = pcT_v[dd, sl]
                hi = pcT_v[dd, sl]
                for k in range(1, 8):
                    ck = pcT_v[3 * k + dd, sl]
                    lo = jnp.minimum(lo, ck)
                    hi = jnp.maximum(hi, ck)
                ext_v[dd, sl] = lo
                ext_v[3 + dd, sl] = hi
            return 0

        lax.fori_loop(0, _Q // _L, _ext_t, 0)

        def _match_j(j, _):
            chunks = []
            for t in range(_Q // _L):
                c = cost_v[j, pl.ds(t * _L, _L)]
                u = used_v[pl.ds(t * _L, _L)]
                chunks.append(jnp.where(u > 0.5, jnp.inf, c))
            best = chunks[0]
            for t in range(1, _Q // _L):
                best = jnp.minimum(best, chunks[t])
            m = jnp.min(best)
            cand = _splat_i(1024)
            for t in range(_Q // _L):
                cand = jnp.minimum(
                    cand, jnp.where(chunks[t] == m, lane + t * _L, 1024)
                )
            idxs = jnp.min(cand)
            plsc.store_scatter(
                used_v, [_splat_i(idxs)], jnp.full((_L,), 1.0, jnp.float32), mask=lane0
            )
            plsc.store_scatter(src_v, [_splat_i(j)], _splat_i(idxs), mask=lane0)
            return 0

        lax.fori_loop(0, _NT, _match_j, 0)

        xz = jnp.float32(0.0)
        bb = jnp.float32(0.0)
        gs = jnp.float32(0.0)
        for cchunk in range(_NT // _L):
            jv = lane + cchunk * _L
            srcv = src_v[pl.ds(cchunk * _L, _L)]
            xz = xz + jnp.sum(plsc.load_gather(y_v, [jv, srcv]))
            bb = bb + jnp.sum(plsc.load_gather(cbb_v, [jv, srcv]))
            inter = jnp.full((_L,), 1.0, jnp.float32)
            vol_s = jnp.full((_L,), 1.0, jnp.float32)
            vol_t = jnp.full((_L,), 1.0, jnp.float32)
            enc = jnp.full((_L,), 1.0, jnp.float32)
            for dd in range(3):
                smn = plsc.load_gather(ext_v, [_splat_i(dd), srcv])
                smx = plsc.load_gather(ext_v, [_splat_i(3 + dd), srcv])
                tmn = tcT_v[dd, pl.ds(cchunk * _L, _L)]
                tmx = tcT_v[dd, pl.ds(cchunk * _L, _L)]
                for k in range(1, 8):
                    ck = tcT_v[3 * k + dd, pl.ds(cchunk * _L, _L)]
                    tmn = jnp.minimum(tmn, ck)
                    tmx = jnp.maximum(tmx, ck)
                inter = inter * jnp.maximum(
                    jnp.minimum(smx, tmx) - jnp.maximum(smn, tmn), 0.0
                )
                vol_s = vol_s * (smx - smn)
                vol_t = vol_t * (tmx - tmn)
                enc = enc * (jnp.maximum(smx, tmx) - jnp.minimum(smn, tmn))
            union = vol_s + vol_t - inter
            g = inter / (union + 1e-7) - (enc - union) / (enc + 1e-7)
            gs = gs + jnp.sum(g)

        out = jnp.where(
            lane0,
            xz,
            jnp.where(lane == 1, bb, jnp.where(lane == 2, gs, 0.0)),
        )
        stage_v[...] = out
        pltpu.sync_copy(stage_v, out_hbm.at[b])


def _sc_partials(x, pbT, tb, lbl, pcT, tcT):
    mesh = plsc.VectorSubcoreMesh(core_axis_name="c", subcore_axis_name="s")
    k = functools.partial(
        pl.kernel,
        mesh=mesh,
        compiler_params=pltpu.CompilerParams(needs_layout_passes=False),
        out_type=jax.ShapeDtypeStruct((_B, _L), jnp.float32),
        scratch_types=[
            pltpu.VMEM((_Q, _C), jnp.float32),
            pltpu.VMEM((6, _Q), jnp.float32),
            pltpu.VMEM((_NT, 6), jnp.float32),
            pltpu.VMEM((_NT,), jnp.int32),
            pltpu.VMEM((24, _Q), jnp.float32),
            pltpu.VMEM((24, _NT), jnp.float32),
            pltpu.VMEM((_NT, _Q), jnp.float32),
            pltpu.VMEM((_NT, _Q), jnp.float32),
            pltpu.VMEM((_NT, _Q), jnp.float32),
            pltpu.VMEM((_Q,), jnp.float32),
            pltpu.VMEM((_NT,), jnp.int32),
            pltpu.VMEM((6, _Q), jnp.float32),
            pltpu.VMEM((_L,), jnp.float32),
            pltpu.SemaphoreType.DMA,
        ],
    )(_sc_body)
    return k(x, pbT, tb, lbl, pcT, tcT)


def _tc_body(xT_ref, part_ref, out_ref):
    x3 = xT_ref[...]
    ce_pos = jnp.sum(jnp.maximum(x3, 0.0) + jnp.log(1.0 + jnp.exp(-jnp.abs(x3))))
    parts = part_ref[...]
    xz = jnp.sum(parts[:, 0:1])
    bbox = jnp.sum(parts[:, 1:2])
    giou_s = jnp.sum(parts[:, 2:3])
    ce = (ce_pos - xz) / (_B * _Q * _C)
    bb = bbox / (_B * _NT * 6)
    gi = 1.0 - giou_s / (_B * _NT)
    out_ref[0] = ce * _WCE + bb * _WBB + gi * _WGI
    out_ref[1] = ce
    out_ref[2] = bb
    out_ref[3] = gi


def kernel(pred_logits, pred_boxes, pred_corners, tgt_labels, tgt_boxes, tgt_corners):
    pbT = jnp.transpose(pred_boxes, (0, 2, 1))
    pcT = jnp.transpose(pred_corners, (0, 2, 3, 1)).reshape(_B, 24, _Q)
    tcT = jnp.transpose(tgt_corners, (0, 2, 3, 1)).reshape(_B, 24, _NT)
    lbl = tgt_labels.astype(jnp.int32)
    parts = _sc_partials(pred_logits, pbT, tgt_boxes, lbl, pcT, tcT)
    xT = jnp.transpose(pred_logits, (0, 2, 1))
    out = pl.pallas_call(
        _tc_body,
        out_shape=jax.ShapeDtypeStruct((4,), jnp.float32),
        out_specs=pl.BlockSpec(memory_space=pltpu.SMEM),
    )(xT, parts)
    return (out[0], out[1], out[2], out[3])


# speedup vs baseline: 4.6769x; 4.6769x over previous
import jax
import jax.numpy as jnp
from jax import lax
from jax.experimental import pallas as pl
from jax.experimental.pallas import tpu as pltpu

_B, _Q, _NT, _C = 8, 256, 32, 32
_WCE, _WBB, _WGI = 1.0, 5.0, 2.0


def _loss_body(xT_ref, pbT_ref, pcT_ref, lbl_ref, tb_ref, tcT_ref, fin_ref, ce_ref, bb_ref, gi_ref):
    x3 = xT_ref[...]
    ce_pos = jnp.sum(jnp.maximum(x3, 0.0) + jnp.log(1.0 + jnp.exp(-jnp.abs(x3))))

    lbl3 = lbl_ref[...]
    y3 = jnp.zeros((_B, _NT, _Q), jnp.float32)
    for g in range(4):
        sub = jnp.clip(lbl3 - 8 * g, 0, 7)
        subB = jnp.broadcast_to(sub, (_B, _NT, _Q))
        part = jnp.take_along_axis(x3[:, 8 * g : 8 * g + 8, :], subB, axis=1)
        y3 = y3 + jnp.where((lbl3 >= 8 * g) & (lbl3 < 8 * g + 8), part, 0.0)

    pb3 = pbT_ref[...]
    tb3 = tb_ref[...]
    cbb3 = jnp.zeros((_B, _NT, _Q), jnp.float32)
    for dd in range(6):
        cbb3 = cbb3 + jnp.abs(pb3[:, dd : dd + 1, :] - tb3[:, :, dd : dd + 1])
    cost3 = -(1.0 / (1.0 + jnp.exp(-y3))) + cbb3

    costT = [jnp.transpose(cost3[:, j, :]) for j in range(_NT)]
    q_iota_s = lax.broadcasted_iota(jnp.int32, (_Q, 1), 0)
    usedT = jnp.zeros((_Q, _B), jnp.float32)
    rows = []
    for j in range(_NT):
        cv = jnp.where(usedT > 0.5, jnp.inf, costT[j])
        m = jnp.min(cv, axis=0, keepdims=True)
        idx = jnp.min(jnp.where(cv == m, q_iota_s, _Q), axis=0, keepdims=True)
        ohqT = jnp.where(q_iota_s == idx, 1.0, 0.0)
        usedT = jnp.maximum(usedT, ohqT)
        rows.append(jnp.transpose(ohqT).reshape(_B, 1, _Q))

    st3 = jnp.concatenate(rows, axis=1)
    xz = jnp.sum(st3 * y3)
    bbox = jnp.sum(st3 * cbb3)

    smin = pcT_ref[:, 0]
    smax = pcT_ref[:, 0]
    for k in range(1, 8):
        ck = pcT_ref[:, k]
        smin = jnp.minimum(smin, ck)
        smax = jnp.maximum(smax, ck)

    giou_s = jnp.float32(0.0)
    for b in range(_B):
        sm6 = jnp.concatenate([smin[b], smax[b]], axis=0)
        mm = lax.dot_general(
            sm6,
            st3[b],
            (((1,), (1,)), ((), ())),
            precision=lax.Precision.HIGHEST,
            preferred_element_type=jnp.float32,
        )
        inter = jnp.float32(1.0)
        vol_s = jnp.float32(1.0)
        vol_t = jnp.float32(1.0)
        enc = jnp.float32(1.0)
        for dd in range(3):
            smn = mm[dd : dd + 1, :]
            smx = mm[3 + dd : 4 + dd, :]
            tmn = tcT_ref[b, dd, 0:1]
            tmx = tcT_ref[b, dd, 0:1]
            for k in range(1, 8):
                ck = tcT_ref[b, dd, k : k + 1]
                tmn = jnp.minimum(tmn, ck)
                tmx = jnp.maximum(tmx, ck)
            inter = inter * jnp.maximum(jnp.minimum(smx, tmx) - jnp.maximum(smn, tmn), 0.0)
            vol_s = vol_s * (smx - smn)
            vol_t = vol_t * (tmx - tmn)
            enc = enc * (jnp.maximum(smx, tmx) - jnp.minimum(smn, tmn))
        union = vol_s + vol_t - inter
        g = inter / (union + 1e-7) - (enc - union) / (enc + 1e-7)
        giou_s = giou_s + jnp.sum(g)

    ce = (ce_pos - xz) / (_B * _Q * _C)
    bb = bbox / (_B * _NT * 6)
    gi = 1.0 - giou_s / (_B * _NT)
    fin_ref[0] = ce * _WCE + bb * _WBB + gi * _WGI
    ce_ref[0] = ce
    bb_ref[0] = bb
    gi_ref[0] = gi


def kernel(pred_logits, pred_boxes, pred_corners, tgt_labels, tgt_boxes, tgt_corners):
    xT = jnp.transpose(pred_logits, (0, 2, 1))
    pbT = jnp.transpose(pred_boxes, (0, 2, 1))
    pcT = jnp.transpose(pred_corners, (0, 2, 3, 1))
    lbl = tgt_labels.astype(jnp.int32).reshape(_B, _NT, 1)
    tcT = jnp.transpose(tgt_corners, (0, 3, 2, 1))
    s = jax.ShapeDtypeStruct((1,), jnp.float32)
    fin, ce, bb, gi = pl.pallas_call(
        _loss_body,
        out_shape=(s, s, s, s),
        out_specs=tuple(pl.BlockSpec(memory_space=pltpu.SMEM) for _ in range(4)),
    )(xT, pbT, pcT, lbl, tgt_boxes, tcT)
    return (fin.reshape(()), ce.reshape(()), bb.reshape(()), gi.reshape(()))


# speedup vs baseline: 6.1049x; 1.3053x over previous
import jax
import jax.numpy as jnp
from jax import lax
from jax.experimental import pallas as pl
from jax.experimental.pallas import tpu as pltpu

_B, _Q, _NT, _C = 8, 256, 32, 32
_WCE, _WBB, _WGI = 1.0, 5.0, 2.0


def _loss_body(xT_ref, pbT_ref, pcT_ref, lbl_ref, tb_ref, tcT_ref, fin_ref, ce_ref, bb_ref, gi_ref):
    x3 = xT_ref[...]
    ce_pos = jnp.sum(jnp.maximum(x3, 0.0) + jnp.log(1.0 + jnp.exp(-jnp.abs(x3))))

    lbl3 = lbl_ref[...]
    y3 = jnp.zeros((_B, _NT, _Q), jnp.float32)
    for g in range(4):
        sub = jnp.clip(lbl3 - 8 * g, 0, 7)
        subB = jnp.broadcast_to(sub, (_B, _NT, _Q))
        part = jnp.take_along_axis(x3[:, 8 * g : 8 * g + 8, :], subB, axis=1)
        y3 = y3 + jnp.where((lbl3 >= 8 * g) & (lbl3 < 8 * g + 8), part, 0.0)

    pb3 = pbT_ref[...]
    tb3 = tb_ref[...]
    cbb3 = jnp.zeros((_B, _NT, _Q), jnp.float32)
    for dd in range(6):
        cbb3 = cbb3 + jnp.abs(pb3[:, dd : dd + 1, :] - tb3[:, :, dd : dd + 1])
    cost3 = -(1.0 / (1.0 + jnp.exp(-y3))) + cbb3

    costT = [jnp.transpose(cost3[:, j, :]) for j in range(_NT)]
    q_iota_s = lax.broadcasted_iota(jnp.int32, (_Q, 1), 0)
    usedT = jnp.zeros((_Q, _B), jnp.float32)
    rows = []
    for j in range(_NT):
        cv = jnp.where(usedT > 0.5, jnp.inf, costT[j])
        m = jnp.min(cv, axis=0, keepdims=True)
        idx = jnp.min(jnp.where(cv == m, q_iota_s, _Q), axis=0, keepdims=True)
        ohqT = jnp.where(q_iota_s == idx, 1.0, 0.0)
        usedT = jnp.maximum(usedT, ohqT)
        rows.append(jnp.transpose(ohqT).reshape(_B, 1, _Q))

    st3 = jnp.concatenate(rows, axis=1)
    xz = jnp.sum(st3 * y3)
    bbox = jnp.sum(st3 * cbb3)

    smin = pcT_ref[:, 0]
    smax = pcT_ref[:, 0]
    for k in range(1, 8):
        ck = pcT_ref[:, k]
        smin = jnp.minimum(smin, ck)
        smax = jnp.maximum(smax, ck)

    giou_s = jnp.float32(0.0)
    for b in range(_B):
        sm6 = jnp.concatenate([smin[b], smax[b]], axis=0)
        mm = lax.dot_general(
            sm6,
            st3[b],
            (((1,), (1,)), ((), ())),
            precision=lax.Precision.HIGHEST,
            preferred_element_type=jnp.float32,
        )
        inter = jnp.float32(1.0)
        vol_s = jnp.float32(1.0)
        vol_t = jnp.float32(1.0)
        enc = jnp.float32(1.0)
        for dd in range(3):
            smn = mm[dd : dd + 1, :]
            smx = mm[3 + dd : 4 + dd, :]
            tmn = tcT_ref[b, dd, 0:1]
            tmx = tcT_ref[b, dd, 0:1]
            for k in range(1, 8):
                ck = tcT_ref[b, dd, k : k + 1]
                tmn = jnp.minimum(tmn, ck)
                tmx = jnp.maximum(tmx, ck)
            inter = inter * jnp.maximum(jnp.minimum(smx, tmx) - jnp.maximum(smn, tmn), 0.0)
            vol_s = vol_s * (smx - smn)
            vol_t = vol_t * (tmx - tmn)
            enc = enc * (jnp.maximum(smx, tmx) - jnp.minimum(smn, tmn))
        union = vol_s + vol_t - inter
        g = inter / (union + 1e-7) - (enc - union) / (enc + 1e-7)
        giou_s = giou_s + jnp.sum(g)

    ce = (ce_pos - xz) / (_B * _Q * _C)
    bb = bbox / (_B * _NT * 6)
    gi = 1.0 - giou_s / (_B * _NT)
    fin_ref[0] = ce * _WCE + bb * _WBB + gi * _WGI
    ce_ref[0] = ce
    bb_ref[0] = bb
    gi_ref[0] = gi


def kernel(pred_logits, pred_boxes, pred_corners, tgt_labels, tgt_boxes, tgt_corners):
    xT = jnp.transpose(pred_logits, (0, 2, 1))
    pbT = jnp.transpose(pred_boxes, (0, 2, 1))
    pcT = jnp.transpose(pred_corners, (0, 2, 3, 1))
    lbl = tgt_labels.astype(jnp.int32).reshape(_B, _NT, 1)
    tcT = jnp.transpose(tgt_corners, (0, 3, 2, 1))
    s = jax.ShapeDtypeStruct((1,), jnp.float32)
    fin, ce, bb, gi = pl.pallas_call(
        _loss_body,
        out_shape=(s, s, s, s),
        out_specs=tuple(pl.BlockSpec(memory_space=pltpu.SMEM) for _ in range(4)),
        compiler_params=pltpu.CompilerParams(allow_input_fusion=[True] * 6),
    )(xT, pbT, pcT, lbl, tgt_boxes, tcT)
    return (fin.reshape(()), ce.reshape(()), bb.reshape(()), gi.reshape(()))
